# baseline (speedup 1.0000x reference)
import jax
import jax.numpy as jnp
from jax import lax
from jax.experimental import pallas as pl
from jax.experimental.pallas import tpu as pltpu
from jax.experimental.pallas import tpu_sc as plsc

BINS = 64
B, C, H, W = 16, 3, 512, 512
ROW = H * W
ROWS_PER_IMG = B * C
NC, NS, LANES = 2, 16, 16
TC_ROWS = 16
SC_ROWS = ROWS_PER_IMG - TC_ROWS
ROWS_PER_TILE = SC_ROWS // NS
CHUNK = 32768
CHUNKS_PER_ROW = ROW // CHUNK
NCHTOT = ROWS_PER_TILE * CHUNKS_PER_ROW
SUBH = 2048
SPAN = ROWS_PER_TILE * ROW
UNROLL = 16


def _sc_hist_body(fake_hbm, real_hbm, out_hbm, buf0, buf1, hist, outbuf):
    c = lax.axis_index("c")
    s = lax.axis_index("s")
    lane = lax.iota(jnp.int32, LANES)
    ones = jnp.ones((LANES,), jnp.float32)
    zeros = jnp.zeros((LANES,), jnp.float32)
    span_base = s * SPAN

    def process(src_hbm, img_base):
        @plsc.parallel_loop(0, ROWS_PER_TILE * SUBH, step=LANES, unroll=4)
        def _(j):
            hist[pl.ds(j, LANES)] = zeros

        def start(ci, buf, sem):
            cic = jnp.minimum(ci, NCHTOT - 1)
            pltpu.make_async_copy(
                src_hbm.at[pl.ds(span_base + cic * CHUNK, CHUNK)], buf, sem
            ).start()

        def wait(buf, sem):
            pltpu.make_async_copy(
                src_hbm.at[pl.ds(span_base, CHUNK)], buf, sem).wait()

        def consume(ci, buf):
            base2 = lane + (ci // CHUNKS_PER_ROW) * SUBH

            @plsc.parallel_loop(0, CHUNK, step=LANES, unroll=UNROLL)
            def _(j):
                x = buf[pl.ds(j, LANES)]
                t = (x + 1.0) * 512.0
                a = (t.astype(jnp.int32) & -16) + base2
                plsc.addupdate_scatter(hist, [a], ones)

        def run(sem0, sem1):
            start(0, buf0, sem0)

            def pair(p, carry):
                ci0 = p * 2
                wait(buf0, sem0)
                start(ci0 + 1, buf1, sem1)
                consume(ci0, buf0)
                wait(buf1, sem1)
                start(ci0 + 2, buf0, sem0)
                consume(ci0 + 1, buf1)
                return carry

            lax.fori_loop(0, NCHTOT // 2, pair, 0)
            wait(buf0, sem0)

        pl.run_scoped(run, pltpu.SemaphoreType.DMA, pltpu.SemaphoreType.DMA)

        for r in range(ROWS_PER_TILE):
            hist[pl.ds(r * SUBH + 63 * LANES, LANES)] = (
                hist[pl.ds(r * SUBH + 63 * LANES, LANES)]
                + hist[pl.ds(r * SUBH + 64 * LANES, LANES)])
            for k in range(BINS // LANES):
                binaddr = (k * LANES + lane) * LANES + r * SUBH
                acc = zeros
                for sub in range(LANES):
                    acc = acc + plsc.load_gather(hist, [binaddr + sub])
                outbuf[pl.ds(k * LANES, LANES)] = acc
            row = img_base + s * ROWS_PER_TILE + r
            pltpu.sync_copy(outbuf, out_hbm.at[pl.ds(row * BINS, BINS)])

    @pl.when(c == 0)
    def _():
        process(fake_hbm, 0)

    @pl.when(c == 1)
    def _():
        process(real_hbm, SC_ROWS)


_sc_hist = pl.kernel(
    _sc_hist_body,
    out_type=jax.ShapeDtypeStruct((2 * SC_ROWS * BINS,), jnp.float32),
    mesh=plsc.VectorSubcoreMesh(
        core_axis_name="c", subcore_axis_name="s",
        num_cores=NC, num_subcores=NS),
    scratch_types=[
        pltpu.VMEM((CHUNK,), jnp.float32),
        pltpu.VMEM((CHUNK,), jnp.float32),
        pltpu.VMEM((ROWS_PER_TILE * SUBH,), jnp.float32),
        pltpu.VMEM((BINS,), jnp.float32),
    ],
    compiler_params=pltpu.CompilerParams(needs_layout_passes=False),
)


def _tc_hist_body(f_ref, r_ref, cf_ref, cr_ref):
    for src, dst in ((f_ref, cf_ref), (r_ref, cr_ref)):
        x = src[0]
        t = (x + 1.0) * 32.0
        i = jnp.minimum(t.astype(jnp.int32), BINS - 1)
        for b in range(BINS):
            dst[0, 0, b] = jnp.sum((i == b).astype(jnp.float32))


_tc_hist = pl.pallas_call(
    _tc_hist_body,
    grid=(TC_ROWS,),
    in_specs=[pl.BlockSpec((1, H, W), lambda i: (SC_ROWS + i, 0, 0)),
              pl.BlockSpec((1, H, W), lambda i: (SC_ROWS + i, 0, 0))],
    out_specs=[pl.BlockSpec((1, 1, BINS), lambda i: (i, 0, 0),
                            memory_space=pltpu.SMEM),
               pl.BlockSpec((1, 1, BINS), lambda i: (i, 0, 0),
                            memory_space=pltpu.SMEM)],
    out_shape=[jax.ShapeDtypeStruct((TC_ROWS, 1, BINS), jnp.float32)] * 2,
)


def _kl_body(cf_ref, cr_ref, out_ref):
    cf = cf_ref[...]
    cr = cr_ref[...]
    hf = cf / (jnp.sum(cf, axis=2, keepdims=True) + 1e-08)
    hr = cr / (jnp.sum(cr, axis=2, keepdims=True) + 1e-08)
    pf = jnp.clip(jnp.mean(hf, axis=0), 1e-08, 1.0)
    pr = jnp.clip(jnp.mean(hr, axis=0), 1e-08, 1.0)
    kl = jnp.sum(pr * (jnp.log(pr) - jnp.log(pf))) / (C * BINS)
    out_ref[0, 0] = kl


_kl = pl.pallas_call(
    _kl_body,
    out_shape=jax.ShapeDtypeStruct((1, 1), jnp.float32),
    out_specs=pl.BlockSpec(memory_space=pltpu.SMEM),
)


@jax.jit
def kernel(fake, real):
    counts = _sc_hist(fake.reshape(-1), real.reshape(-1))
    tc_cf, tc_cr = _tc_hist(fake.reshape(ROWS_PER_IMG, H, W),
                            real.reshape(ROWS_PER_IMG, H, W))
    cf = jnp.concatenate(
        [counts[: SC_ROWS * BINS].reshape(SC_ROWS, BINS),
         tc_cf.reshape(TC_ROWS, BINS)])
    cr = jnp.concatenate(
        [counts[SC_ROWS * BINS:].reshape(SC_ROWS, BINS),
         tc_cr.reshape(TC_ROWS, BINS)])
    return _kl(cf.reshape(B, C, BINS), cr.reshape(B, C, BINS))[0, 0]


# speedup vs baseline: 2.0051x; 2.0051x over previous
import jax
import jax.numpy as jnp
from jax import lax
from jax.experimental import pallas as pl
from jax.experimental.pallas import tpu as pltpu
from jax.experimental.pallas import tpu_sc as plsc

BINS = 64
B, C, H, W = 16, 3, 512, 512
ROW = H * W
ROWS_PER_IMG = B * C
NC, NS, LANES = 2, 16, 16
ROWS_PER_TILE = ROWS_PER_IMG // NS
CHUNK = 32768
CHUNKS_PER_ROW = ROW // CHUNK
NCHTOT = ROWS_PER_TILE * CHUNKS_PER_ROW
SUBH = 2048
SPAN = ROWS_PER_TILE * ROW
UNROLL = 16


def _sc_hist_body(fake_hbm, real_hbm, out_hbm, buf0, buf1, hist, outbuf):
    c = lax.axis_index("c")
    s = lax.axis_index("s")
    lane = lax.iota(jnp.int32, LANES)
    ones = jnp.ones((LANES,), jnp.float32)
    zeros = jnp.zeros((LANES,), jnp.float32)
    span_base = s * SPAN

    def process(src_hbm, img_base):
        @plsc.parallel_loop(0, ROWS_PER_TILE * SUBH, step=LANES, unroll=4)
        def _(j):
            hist[pl.ds(j, LANES)] = zeros

        def start(ci, buf, sem):
            cic = jnp.minimum(ci, NCHTOT - 1)
            pltpu.make_async_copy(
                src_hbm.at[pl.ds(span_base + cic * CHUNK, CHUNK)], buf, sem
            ).start()

        def wait(buf, sem):
            pltpu.make_async_copy(
                src_hbm.at[pl.ds(span_base, CHUNK)], buf, sem).wait()

        def consume(ci, buf):
            base2 = lane + (ci // CHUNKS_PER_ROW) * SUBH

            @plsc.parallel_loop(0, CHUNK, step=LANES, unroll=UNROLL)
            def _(j):
                x = buf[pl.ds(j, LANES)]
                t = (x + 1.0) * 512.0
                a = (t.astype(jnp.int32) & -16) + base2
                plsc.addupdate_scatter(hist, [a], ones)

        def run(sem0, sem1):
            start(0, buf0, sem0)

            def pair(p, carry):
                ci0 = p * 2
                wait(buf0, sem0)
                start(ci0 + 1, buf1, sem1)
                consume(ci0, buf0)
                wait(buf1, sem1)
                start(ci0 + 2, buf0, sem0)
                consume(ci0 + 1, buf1)
                return carry

            lax.fori_loop(0, NCHTOT // 2, pair, 0)
            wait(buf0, sem0)

        pl.run_scoped(run, pltpu.SemaphoreType.DMA, pltpu.SemaphoreType.DMA)

        for r in range(ROWS_PER_TILE):
            hist[pl.ds(r * SUBH + 63 * LANES, LANES)] = (
                hist[pl.ds(r * SUBH + 63 * LANES, LANES)]
                + hist[pl.ds(r * SUBH + 64 * LANES, LANES)])
            for k in range(BINS // LANES):
                binaddr = (k * LANES + lane) * LANES + r * SUBH
                acc = zeros
                for sub in range(LANES):
                    acc = acc + plsc.load_gather(hist, [binaddr + sub])
                outbuf[pl.ds(k * LANES, LANES)] = acc
            row = img_base + s * ROWS_PER_TILE + r
            pltpu.sync_copy(outbuf, out_hbm.at[pl.ds(row * BINS, BINS)])

    @pl.when(c == 0)
    def _():
        process(fake_hbm, 0)

    @pl.when(c == 1)
    def _():
        process(real_hbm, ROWS_PER_IMG)


_sc_hist = pl.kernel(
    _sc_hist_body,
    out_type=jax.ShapeDtypeStruct((2 * ROWS_PER_IMG * BINS,), jnp.float32),
    mesh=plsc.VectorSubcoreMesh(
        core_axis_name="c", subcore_axis_name="s",
        num_cores=NC, num_subcores=NS),
    scratch_types=[
        pltpu.VMEM((CHUNK,), jnp.float32),
        pltpu.VMEM((CHUNK,), jnp.float32),
        pltpu.VMEM((ROWS_PER_TILE * SUBH,), jnp.float32),
        pltpu.VMEM((BINS,), jnp.float32),
    ],
    compiler_params=pltpu.CompilerParams(needs_layout_passes=False),
)


def _kl_body(cf_ref, cr_ref, out_ref):
    cf = cf_ref[...]
    cr = cr_ref[...]
    hf = cf / (jnp.sum(cf, axis=2, keepdims=True) + 1e-08)
    hr = cr / (jnp.sum(cr, axis=2, keepdims=True) + 1e-08)
    pf = jnp.clip(jnp.mean(hf, axis=0), 1e-08, 1.0)
    pr = jnp.clip(jnp.mean(hr, axis=0), 1e-08, 1.0)
    kl = jnp.sum(pr * (jnp.log(pr) - jnp.log(pf))) / (C * BINS)
    out_ref[0, 0] = kl


_kl = pl.pallas_call(
    _kl_body,
    out_shape=jax.ShapeDtypeStruct((1, 1), jnp.float32),
    out_specs=pl.BlockSpec(memory_space=pltpu.SMEM),
)


@jax.jit
def kernel(fake, real):
    counts = _sc_hist(fake.reshape(-1), real.reshape(-1))
    cf = counts[: ROWS_PER_IMG * BINS].reshape(B, C, BINS)
    cr = counts[ROWS_PER_IMG * BINS:].reshape(B, C, BINS)
    return _kl(cf, cr)[0, 0]
